# baseline (speedup 1.0000x reference)
import functools

import jax
import jax.numpy as jnp
from jax import lax
from jax.experimental import pallas as pl
from jax.experimental.pallas import tpu as pltpu
from jax.experimental.pallas import tpu_sc as plsc

N = 10000
E = 320000
IN_DIM = 128
EDGE_DIM = 16
HID = 32
EMB = 128
NG = 16
CFGS = [(HID, 4, HID, True), (HID * 4, 4, HID, True), (HID * 4, 4, HID, True), (HID * 4, 1, HID, False)]

NC = 2
NS = 16
NW = NC * NS
LANES = 16
P = 16
KA = 112
KB = 96
E2 = E + N
T = 10752
NCHA = T // KA
NCHB = T // KB
E2P = T * NW
NP = 10240
RPT = NP // NS
K0 = 80
T0 = E // NW

_MESH = plsc.VectorSubcoreMesh(core_axis_name="c", subcore_axis_name="s",
                               num_cores=NC, num_subcores=NS)

_f32 = jnp.float32
_i32 = jnp.int32


def _matmul_bias_kernel(x_ref, w_ref, b_ref, o_ref):
    o_ref[...] = jnp.dot(x_ref[...], w_ref[...],
                         preferred_element_type=_f32) + b_ref[...]


def _dense(x, W, b, row_block=None):
    m, k = x.shape
    n = W.shape[1]
    if row_block is None:
        return pl.pallas_call(
            _matmul_bias_kernel,
            out_shape=jax.ShapeDtypeStruct((m, n), _f32),
        )(x, W, b[None, :])
    assert m % row_block == 0
    return pl.pallas_call(
        _matmul_bias_kernel,
        grid=(m // row_block,),
        in_specs=[
            pl.BlockSpec((row_block, k), lambda i: (i, 0)),
            pl.BlockSpec((k, n), lambda i: (0, 0)),
            pl.BlockSpec((1, n), lambda i: (0, 0)),
        ],
        out_specs=pl.BlockSpec((row_block, n), lambda i: (i, 0)),
        out_shape=jax.ShapeDtypeStruct((m, n), _f32),
    )(x, W, b[None, :])


def _loopattr_kernel(ea0_ref, ea1_ref, dg0_ref, dg1_ref, o_ref):
    deg = dg0_ref[0:N, 0:1] + dg1_ref[0:N, 0:1]
    o_ref[...] = (ea0_ref[0:N] + ea1_ref[0:N]) / jnp.maximum(deg, 1.0)


def _loopattr(ea0, ea1, dg0, dg1):
    return pl.pallas_call(
        _loopattr_kernel,
        out_shape=jax.ShapeDtypeStruct((N, EDGE_DIM), _f32),
    )(ea0, ea1, dg0, dg1)


def _gelu(v):
    return v * 0.5 * (1.0 + lax.erf(v * 0.7071067811865476))


def _ln_gelu(v, g, b):
    mu = jnp.mean(v, axis=-1, keepdims=True)
    var = jnp.mean((v - mu) ** 2, axis=-1, keepdims=True)
    return _gelu((v - mu) / jnp.sqrt(var + 1e-5) * g + b)


def _post_id_kernel(o0_ref, o1_ref, bias_ref, res_ref, g_ref, b_ref, o_ref):
    v = o0_ref[0:N] + o1_ref[0:N] + bias_ref[...] + res_ref[...]
    o_ref[...] = _ln_gelu(v, g_ref[...], b_ref[...])


def _post_proj_kernel(o0_ref, o1_ref, bias_ref, res_ref, w_ref, rb_ref,
                      g_ref, b_ref, o_ref):
    r = jnp.dot(res_ref[...], w_ref[...], preferred_element_type=_f32) + rb_ref[...]
    v = o0_ref[0:N] + o1_ref[0:N] + bias_ref[...] + r
    o_ref[...] = _ln_gelu(v, g_ref[...], b_ref[...])


def _post(out0, out1, bias, h_res, rp, g, b):
    n = out0.shape[1]
    if rp is None:
        return pl.pallas_call(
            _post_id_kernel,
            out_shape=jax.ShapeDtypeStruct((N, n), _f32),
        )(out0, out1, bias[None, :], h_res, g[None, :], b[None, :])
    return pl.pallas_call(
        _post_proj_kernel,
        out_shape=jax.ShapeDtypeStruct((N, n), _f32),
    )(out0, out1, bias[None, :], h_res, rp["W"], rp["b"][None, :],
      g[None, :], b[None, :])


def _denmerge_kernel(d0_ref, d1_ref, o_ref):
    o_ref[...] = d0_ref[...] + d1_ref[...]


def _denmerge(d0, d1):
    return pl.pallas_call(
        _denmerge_kernel,
        out_shape=jax.ShapeDtypeStruct((NP, P), _f32),
    )(d0, d1)


def _pool_head_kernel(h_ref, batch_ref, apw_ref, apb_ref,
                      o1w_ref, o1b_ref, o2w_ref, o2b_ref, o_ref):
    h = h_ref[...]
    att = jax.nn.sigmoid(jnp.dot(h, apw_ref[...],
                                 preferred_element_type=_f32) + apb_ref[...])
    hw = h * att
    onehot = (batch_ref[...] == lax.broadcasted_iota(_i32, (N, NG), 1)
              ).astype(_f32)
    pooled = jnp.dot(onehot.T, hw, preferred_element_type=_f32)
    e = jnp.maximum(jnp.dot(pooled, o1w_ref[...],
                            preferred_element_type=_f32) + o1b_ref[...], 0.0)
    e = jnp.dot(e, o2w_ref[...], preferred_element_type=_f32) + o2b_ref[...]
    nrm = jnp.sqrt(jnp.sum(e * e, axis=-1, keepdims=True))
    o_ref[...] = e / jnp.maximum(nrm, 1e-12)


def _pool_head(h, batch_i32, p):
    return pl.pallas_call(
        _pool_head_kernel,
        out_shape=jax.ShapeDtypeStruct((NG, EMB), _f32),
    )(h, batch_i32[:, None], p["ap_W"], p["ap_b"][None, :],
      p["o1_W"], p["o1_b"][None, :], p["o2_W"], p["o2_b"][None, :])


def _worker_id():
    return lax.axis_index("c") * NS + lax.axis_index("s")


def _zero_table(tab, zrows_hbm):
    s = lax.axis_index("s")
    pltpu.sync_copy(zrows_hbm, tab.at[pl.ds(s * RPT, RPT)])
    plsc.subcore_barrier()


def _pass0_body(ea_hbm, dst_hbm, ones_hbm, zrows_hbm,
                ea0, ea1, dg0, dg1,
                eatab, degtab, eab, onesb, didx):
    c = lax.axis_index("c")
    s = lax.axis_index("s")
    w = _worker_id()
    _zero_table(eatab, zrows_hbm)
    _zero_table(degtab, zrows_hbm)
    pltpu.sync_copy(ones_hbm, onesb)
    base0 = w * T0

    def chunk(i, _):
        base = base0 + i * K0
        pltpu.sync_copy(dst_hbm.at[pl.ds(base, K0)], didx)
        pltpu.sync_copy(ea_hbm.at[pl.ds(base, K0)], eab)
        pltpu.sync_copy(eab, eatab.at[didx], add=True)
        pltpu.sync_copy(onesb, degtab.at[didx], add=True)
        return 0

    lax.fori_loop(0, T0 // K0, chunk, 0)
    plsc.subcore_barrier()
    rb = s * RPT

    @pl.when(c == 0)
    def _():
        pltpu.sync_copy(eatab.at[pl.ds(rb, RPT)], ea0.at[pl.ds(rb, RPT)])
        pltpu.sync_copy(degtab.at[pl.ds(rb, RPT)], dg0.at[pl.ds(rb, RPT)])

    @pl.when(c == 1)
    def _():
        pltpu.sync_copy(eatab.at[pl.ds(rb, RPT)], ea1.at[pl.ds(rb, RPT)])
        pltpu.sync_copy(degtab.at[pl.ds(rb, RPT)], dg1.at[pl.ds(rb, RPT)])


@functools.partial(
    pl.kernel,
    out_type=(jax.ShapeDtypeStruct((NP, EDGE_DIM), _f32),
              jax.ShapeDtypeStruct((NP, EDGE_DIM), _f32),
              jax.ShapeDtypeStruct((NP, EDGE_DIM), _f32),
              jax.ShapeDtypeStruct((NP, EDGE_DIM), _f32)),
    mesh=_MESH,
    compiler_params=pltpu.CompilerParams(needs_layout_passes=False, use_tc_tiling_on_sc=False),
    scratch_types=[
        pltpu.VMEM_SHARED((NP, EDGE_DIM), _f32),
        pltpu.VMEM_SHARED((NP, EDGE_DIM), _f32),
        pltpu.VMEM((K0, EDGE_DIM), _f32),
        pltpu.VMEM((K0, EDGE_DIM), _f32),
        pltpu.VMEM((K0,), _i32),
    ],
)
def _pass0(*args):
    _pass0_body(*args)


def _passA_body(H, HC, xl_hbm, xr_hbm, ee_hbm, src3_hbm, dst3_hbm, att_hbm,
                zrows_hbm, expl, den0, den1,
                dentab, sidxv, didxv, xlb0, xrb0, eeb0, xlb1, xrb1, eeb1,
                lg0, lg1, attv_ref,
                gx0, gr0, ge0, gx1, gr1, ge1, se0, sd0, se1, sd1):
    CH16 = HC // LANES
    c = lax.axis_index("c")
    s = lax.axis_index("s")
    w = _worker_id()
    _zero_table(dentab, zrows_hbm)
    pltpu.sync_copy(att_hbm, attv_ref)
    pltpu.sync_copy(src3_hbm.at[w], sidxv)
    pltpu.sync_copy(dst3_hbm.at[w], didxv)
    attv = [attv_ref[pl.ds(LANES * j, LANES)] for j in range(CH16)]
    lane = lax.iota(_i32, LANES)
    onehot = [jnp.where(lane == h, 1.0, 0.0).astype(_f32) for h in range(H)]
    base0 = w * T

    def issue(i, xlb, xrb, eeb, s1, s2, s3):
        pltpu.async_copy(xl_hbm.at[sidxv.at[i]], xlb, s1)
        pltpu.async_copy(xr_hbm.at[didxv.at[i]], xrb, s2)
        pltpu.async_copy(ee_hbm.at[pl.ds(base0 + i * KA, KA)], eeb, s3)

    def wait(i, xlb, xrb, eeb, s1, s2, s3):
        pltpu.make_async_copy(xl_hbm.at[sidxv.at[i]], xlb, s1).wait()
        pltpu.make_async_copy(xr_hbm.at[didxv.at[i]], xrb, s2).wait()
        pltpu.make_async_copy(ee_hbm.at[pl.ds(base0 + i * KA, KA)], eeb, s3).wait()

    def compute(i, g, xlb, xrb, eeb, lg, sste, sstd):
        base = base0 + i * KA

        @pl.when(g > 0)
        def _():
            pltpu.make_async_copy(lg, expl.at[pl.ds(base, KA)], sste).wait()
            pltpu.make_async_copy(lg, dentab.at[didxv.at[i]], sstd).wait()

        def edge(k, _):
            acc = []
            for j in range(CH16):
                sl = pl.ds(LANES * j, LANES)
                v = xlb[k, sl] + xrb[k, sl] + eeb[k, sl]
                v = jnp.maximum(v, 0.2 * v)
                acc.append(v * attv[j])
            row = jnp.zeros((LANES,), _f32)
            for h in range(H):
                sh = jnp.sum(acc[2 * h] + acc[2 * h + 1])
                row = row + onehot[h] * sh
            ex = jnp.exp(row)
            ex = jnp.where(base + k < E2, ex, jnp.zeros((LANES,), _f32))
            lg[k, :] = ex
            return 0

        lax.fori_loop(0, KA, edge, 0, unroll=2)
        pltpu.async_copy(lg, expl.at[pl.ds(base, KA)], sste)
        pltpu.async_copy(lg, dentab.at[didxv.at[i]], sstd, add=True)

    issue(0, xlb0, xrb0, eeb0, gx0, gr0, ge0)

    def pair(g, _):
        i0 = 2 * g
        i1 = i0 + 1
        issue(i1, xlb1, xrb1, eeb1, gx1, gr1, ge1)
        wait(i0, xlb0, xrb0, eeb0, gx0, gr0, ge0)
        compute(i0, g, xlb0, xrb0, eeb0, lg0, se0, sd0)

        @pl.when(i1 + 1 < NCHA)
        def _():
            issue(i1 + 1, xlb0, xrb0, eeb0, gx0, gr0, ge0)

        wait(i1, xlb1, xrb1, eeb1, gx1, gr1, ge1)
        compute(i1, g, xlb1, xrb1, eeb1, lg1, se1, sd1)
        return 0

    lax.fori_loop(0, NCHA // 2, pair, 0)
    pltpu.make_async_copy(lg0, expl.at[pl.ds(base0, KA)], se0).wait()
    pltpu.make_async_copy(lg0, dentab.at[didxv.at[0]], sd0).wait()
    pltpu.make_async_copy(lg1, expl.at[pl.ds(base0, KA)], se1).wait()
    pltpu.make_async_copy(lg1, dentab.at[didxv.at[0]], sd1).wait()
    plsc.subcore_barrier()
    rb = s * RPT

    @pl.when(c == 0)
    def _():
        pltpu.sync_copy(dentab.at[pl.ds(rb, RPT)], den0.at[pl.ds(rb, RPT)])

    @pl.when(c == 1)
    def _():
        pltpu.sync_copy(dentab.at[pl.ds(rb, RPT)], den1.at[pl.ds(rb, RPT)])


def _make_passA(H, HC):
    return pl.kernel(
        functools.partial(_passA_body, H, HC),
        out_type=(jax.ShapeDtypeStruct((E2P, P), _f32),
                  jax.ShapeDtypeStruct((NP, P), _f32),
                  jax.ShapeDtypeStruct((NP, P), _f32)),
        mesh=_MESH,
        compiler_params=pltpu.CompilerParams(needs_layout_passes=False, use_tc_tiling_on_sc=False),
        scratch_types=(
            [pltpu.VMEM_SHARED((NP, P), _f32),
             pltpu.VMEM((NCHA, KA), _i32),
             pltpu.VMEM((NCHA, KA), _i32)]
            + [pltpu.VMEM((KA, HC), _f32)] * 6
            + [pltpu.VMEM((KA, P), _f32)] * 2
            + [pltpu.VMEM((HC,), _f32)]
            + [pltpu.SemaphoreType.DMA] * 10
        ),
    )


def _passB_body(H, HC, xl_hbm, idx3_hbm, expl_hbm, den_hbm,
                zrows_hbm, out0, out1,
                outtab, idxb0, idxb1, xlb0, elg0, db0,
                xlb1, elg1, db1, msg,
                ga0, gb0, gc0, ga1, gb1, gc1):
    CH16 = HC // LANES
    c = lax.axis_index("c")
    s = lax.axis_index("s")
    w = _worker_id()
    _zero_table(outtab, zrows_hbm)
    base0 = w * T

    def issue(i, idxb, xlb, elg, db, s1, s2, s3):
        pltpu.async_copy(xl_hbm.at[idxb.at[0]], xlb, s1)
        pltpu.async_copy(expl_hbm.at[pl.ds(base0 + i * KB, KB)], elg, s2)
        pltpu.async_copy(den_hbm.at[idxb.at[1]], db, s3)

    def wait(i, idxb, xlb, elg, db, s1, s2, s3):
        pltpu.make_async_copy(xl_hbm.at[idxb.at[0]], xlb, s1).wait()
        pltpu.make_async_copy(expl_hbm.at[pl.ds(base0 + i * KB, KB)], elg, s2).wait()
        pltpu.make_async_copy(den_hbm.at[idxb.at[1]], db, s3).wait()

    def compute(i, idxb, xlb, elg, db):
        def edge(k, _):
            wv = elg[k, :] / db[k, :]
            for j in range(CH16):
                sl = pl.ds(LANES * j, LANES)
                sp = jnp.full((LANES,), wv[j // 2 if H > 1 else 0], _f32)
                msg[k, sl] = xlb[k, sl] * sp
            return 0

        lax.fori_loop(0, KB, edge, 0, unroll=2)
        pltpu.sync_copy(msg, outtab.at[idxb.at[1]], add=True)

    pltpu.sync_copy(idx3_hbm.at[w, 0], idxb0)
    pltpu.sync_copy(idx3_hbm.at[w, 1], idxb1)
    issue(0, idxb0, xlb0, elg0, db0, ga0, gb0, gc0)

    def pair(g, _):
        i0 = 2 * g
        i1 = i0 + 1
        issue(i1, idxb1, xlb1, elg1, db1, ga1, gb1, gc1)
        wait(i0, idxb0, xlb0, elg0, db0, ga0, gb0, gc0)
        compute(i0, idxb0, xlb0, elg0, db0)

        @pl.when(i0 + 2 < NCHB)
        def _():
            pltpu.sync_copy(idx3_hbm.at[w, i0 + 2], idxb0)
            issue(i0 + 2, idxb0, xlb0, elg0, db0, ga0, gb0, gc0)

        wait(i1, idxb1, xlb1, elg1, db1, ga1, gb1, gc1)
        compute(i1, idxb1, xlb1, elg1, db1)

        @pl.when(i1 + 2 < NCHB)
        def _():
            pltpu.sync_copy(idx3_hbm.at[w, i1 + 2], idxb1)

        return 0

    lax.fori_loop(0, NCHB // 2, pair, 0)
    plsc.subcore_barrier()
    rb = s * RPT

    @pl.when(c == 0)
    def _():
        pltpu.sync_copy(outtab.at[pl.ds(rb, RPT)], out0.at[pl.ds(rb, RPT)])

    @pl.when(c == 1)
    def _():
        pltpu.sync_copy(outtab.at[pl.ds(rb, RPT)], out1.at[pl.ds(rb, RPT)])


def _make_passB(H, HC):
    return pl.kernel(
        functools.partial(_passB_body, H, HC),
        out_type=(jax.ShapeDtypeStruct((NP, HC), _f32),
                  jax.ShapeDtypeStruct((NP, HC), _f32)),
        mesh=_MESH,
        compiler_params=pltpu.CompilerParams(needs_layout_passes=False, use_tc_tiling_on_sc=False),
        scratch_types=(
            [pltpu.VMEM_SHARED((NP, HC), _f32),
             pltpu.VMEM((2, KB), _i32),
             pltpu.VMEM((2, KB), _i32)]
            + [pltpu.VMEM((KB, HC), _f32), pltpu.VMEM((KB, P), _f32),
               pltpu.VMEM((KB, P), _f32)] * 2
            + [pltpu.VMEM((KB, HC), _f32)]
            + [pltpu.SemaphoreType.DMA] * 6
        ),
    )


_PASSA = {(4, 128): _make_passA(4, 128), (1, 32): _make_passA(1, 32)}
_PASSB = {(4, 128): _make_passB(4, 128), (1, 32): _make_passB(1, 32)}


def kernel(x, edge_index, edge_attr, batch, params):
    src = edge_index[0].astype(_i32)
    dst = edge_index[1].astype(_i32)
    batch_i32 = batch.astype(_i32)

    zrows16 = jnp.zeros((RPT, P), _f32)
    ones0 = jnp.zeros((K0, EDGE_DIM), _f32).at[:, 0].set(1.0)

    ea0, ea1, dg0, dg1 = _pass0(edge_attr, dst, ones0, jnp.zeros((RPT, EDGE_DIM), _f32))
    loop_attr = _loopattr(ea0, ea1, dg0, dg1)

    loop = jnp.arange(N, dtype=_i32)
    padi = jnp.zeros((E2P - E2,), _i32)
    src2 = jnp.concatenate([src, loop, padi])
    dst2 = jnp.concatenate([dst, loop, padi])
    src3a = src2.reshape(NW, NCHA, KA)
    dst3a = dst2.reshape(NW, NCHA, KA)
    idx3b = jnp.concatenate([src2.reshape(NW, NCHB, 1, KB),
                             dst2.reshape(NW, NCHB, 1, KB)], axis=2)
    ea2 = jnp.concatenate([edge_attr, loop_attr,
                           jnp.zeros((E2P - E2, EDGE_DIM), _f32)], axis=0)

    h = _dense(x, params["in_W"], params["in_b"])
    for i, (cin, H, C, concat) in enumerate(CFGS):
        HC = H * C
        p = params["gat"][i]
        h_res = h
        xl = _dense(h, p["Wl"], p["bl"])
        xr = _dense(h, p["Wr"], p["br"])
        eemb = _dense(ea2, p["We"], jnp.zeros((HC,), _f32), row_block=4096)
        attf = p["att"].reshape(-1)
        expl, den0, den1 = _PASSA[(H, HC)](xl, xr, eemb, src3a, dst3a, attf, zrows16)
        den = _denmerge(den0, den1)
        out0, out1 = _PASSB[(H, HC)](xl, idx3b, expl, den,
                                     jnp.zeros((RPT, HC), _f32))
        h = _post(out0, out1, p["bias"], h_res, params["res"][i],
                  params["ln"][i]["g"], params["ln"][i]["b"])

    return _pool_head(h, batch_i32, params)


# speedup vs baseline: 1.0155x; 1.0155x over previous
import functools

import jax
import jax.numpy as jnp
from jax import lax
from jax.experimental import pallas as pl
from jax.experimental.pallas import tpu as pltpu
from jax.experimental.pallas import tpu_sc as plsc

N = 10000
E = 320000
IN_DIM = 128
EDGE_DIM = 16
HID = 32
EMB = 128
NG = 16
CFGS = [(HID, 4, HID, True), (HID * 4, 4, HID, True), (HID * 4, 4, HID, True), (HID * 4, 1, HID, False)]

NC = 2
NS = 16
NW = NC * NS
LANES = 16
P = 16
KA = 128
KB = 96
E2 = E + N
T = 10752
NCHA = T // KA
NCHB = T // KB
E2P = T * NW
NP = 10240
RPT = NP // NS
K0 = 80
T0 = E // NW

_MESH = plsc.VectorSubcoreMesh(core_axis_name="c", subcore_axis_name="s",
                               num_cores=NC, num_subcores=NS)

_f32 = jnp.float32
_i32 = jnp.int32


def _matmul_bias_kernel(x_ref, w_ref, b_ref, o_ref):
    o_ref[...] = jnp.dot(x_ref[...], w_ref[...],
                         preferred_element_type=_f32) + b_ref[...]


def _dense(x, W, b, row_block=None):
    m, k = x.shape
    n = W.shape[1]
    if row_block is None:
        return pl.pallas_call(
            _matmul_bias_kernel,
            out_shape=jax.ShapeDtypeStruct((m, n), _f32),
        )(x, W, b[None, :])
    assert m % row_block == 0
    return pl.pallas_call(
        _matmul_bias_kernel,
        grid=(m // row_block,),
        in_specs=[
            pl.BlockSpec((row_block, k), lambda i: (i, 0)),
            pl.BlockSpec((k, n), lambda i: (0, 0)),
            pl.BlockSpec((1, n), lambda i: (0, 0)),
        ],
        out_specs=pl.BlockSpec((row_block, n), lambda i: (i, 0)),
        out_shape=jax.ShapeDtypeStruct((m, n), _f32),
    )(x, W, b[None, :])


def _loopattr_kernel(ea0_ref, ea1_ref, dg0_ref, dg1_ref, o_ref):
    deg = dg0_ref[0:N, 0:1] + dg1_ref[0:N, 0:1]
    o_ref[...] = (ea0_ref[0:N] + ea1_ref[0:N]) / jnp.maximum(deg, 1.0)


def _loopattr(ea0, ea1, dg0, dg1):
    return pl.pallas_call(
        _loopattr_kernel,
        out_shape=jax.ShapeDtypeStruct((N, EDGE_DIM), _f32),
    )(ea0, ea1, dg0, dg1)


def _gelu(v):
    return v * 0.5 * (1.0 + lax.erf(v * 0.7071067811865476))


def _ln_gelu(v, g, b):
    mu = jnp.mean(v, axis=-1, keepdims=True)
    var = jnp.mean((v - mu) ** 2, axis=-1, keepdims=True)
    return _gelu((v - mu) / jnp.sqrt(var + 1e-5) * g + b)


def _post_id_kernel(o0_ref, o1_ref, bias_ref, res_ref, g_ref, b_ref, o_ref):
    v = o0_ref[0:N] + o1_ref[0:N] + bias_ref[...] + res_ref[...]
    o_ref[...] = _ln_gelu(v, g_ref[...], b_ref[...])


def _post_proj_kernel(o0_ref, o1_ref, bias_ref, res_ref, w_ref, rb_ref,
                      g_ref, b_ref, o_ref):
    r = jnp.dot(res_ref[...], w_ref[...], preferred_element_type=_f32) + rb_ref[...]
    v = o0_ref[0:N] + o1_ref[0:N] + bias_ref[...] + r
    o_ref[...] = _ln_gelu(v, g_ref[...], b_ref[...])


def _post(out0, out1, bias, h_res, rp, g, b):
    n = out0.shape[1]
    if rp is None:
        return pl.pallas_call(
            _post_id_kernel,
            out_shape=jax.ShapeDtypeStruct((N, n), _f32),
        )(out0, out1, bias[None, :], h_res, g[None, :], b[None, :])
    return pl.pallas_call(
        _post_proj_kernel,
        out_shape=jax.ShapeDtypeStruct((N, n), _f32),
    )(out0, out1, bias[None, :], h_res, rp["W"], rp["b"][None, :],
      g[None, :], b[None, :])


def _denmerge_kernel(d0_ref, d1_ref, o_ref):
    o_ref[...] = d0_ref[...] + d1_ref[...]


def _denmerge(d0, d1):
    return pl.pallas_call(
        _denmerge_kernel,
        out_shape=jax.ShapeDtypeStruct((NP, P), _f32),
    )(d0, d1)


def _pool_head_kernel(h_ref, batch_ref, apw_ref, apb_ref,
                      o1w_ref, o1b_ref, o2w_ref, o2b_ref, o_ref):
    h = h_ref[...]
    att = jax.nn.sigmoid(jnp.dot(h, apw_ref[...],
                                 preferred_element_type=_f32) + apb_ref[...])
    hw = h * att
    onehot = (batch_ref[...] == lax.broadcasted_iota(_i32, (N, NG), 1)
              ).astype(_f32)
    pooled = jnp.dot(onehot.T, hw, preferred_element_type=_f32)
    e = jnp.maximum(jnp.dot(pooled, o1w_ref[...],
                            preferred_element_type=_f32) + o1b_ref[...], 0.0)
    e = jnp.dot(e, o2w_ref[...], preferred_element_type=_f32) + o2b_ref[...]
    nrm = jnp.sqrt(jnp.sum(e * e, axis=-1, keepdims=True))
    o_ref[...] = e / jnp.maximum(nrm, 1e-12)


def _pool_head(h, batch_i32, p):
    return pl.pallas_call(
        _pool_head_kernel,
        out_shape=jax.ShapeDtypeStruct((NG, EMB), _f32),
    )(h, batch_i32[:, None], p["ap_W"], p["ap_b"][None, :],
      p["o1_W"], p["o1_b"][None, :], p["o2_W"], p["o2_b"][None, :])


def _worker_id():
    return lax.axis_index("c") * NS + lax.axis_index("s")


def _zero_table(tab, zrows_hbm):
    s = lax.axis_index("s")
    pltpu.sync_copy(zrows_hbm, tab.at[pl.ds(s * RPT, RPT)])
    plsc.subcore_barrier()


def _pass0_body(ea_hbm, dst_hbm, ones_hbm, zrows_hbm,
                ea0, ea1, dg0, dg1,
                eatab, degtab, eab, onesb, didx):
    c = lax.axis_index("c")
    s = lax.axis_index("s")
    w = _worker_id()
    _zero_table(eatab, zrows_hbm)
    _zero_table(degtab, zrows_hbm)
    pltpu.sync_copy(ones_hbm, onesb)
    base0 = w * T0

    def chunk(i, _):
        base = base0 + i * K0
        pltpu.sync_copy(dst_hbm.at[pl.ds(base, K0)], didx)
        pltpu.sync_copy(ea_hbm.at[pl.ds(base, K0)], eab)
        pltpu.sync_copy(eab, eatab.at[didx], add=True)
        pltpu.sync_copy(onesb, degtab.at[didx], add=True)
        return 0

    lax.fori_loop(0, T0 // K0, chunk, 0)
    plsc.subcore_barrier()
    rb = s * RPT

    @pl.when(c == 0)
    def _():
        pltpu.sync_copy(eatab.at[pl.ds(rb, RPT)], ea0.at[pl.ds(rb, RPT)])
        pltpu.sync_copy(degtab.at[pl.ds(rb, RPT)], dg0.at[pl.ds(rb, RPT)])

    @pl.when(c == 1)
    def _():
        pltpu.sync_copy(eatab.at[pl.ds(rb, RPT)], ea1.at[pl.ds(rb, RPT)])
        pltpu.sync_copy(degtab.at[pl.ds(rb, RPT)], dg1.at[pl.ds(rb, RPT)])


@functools.partial(
    pl.kernel,
    out_type=(jax.ShapeDtypeStruct((NP, EDGE_DIM), _f32),
              jax.ShapeDtypeStruct((NP, EDGE_DIM), _f32),
              jax.ShapeDtypeStruct((NP, EDGE_DIM), _f32),
              jax.ShapeDtypeStruct((NP, EDGE_DIM), _f32)),
    mesh=_MESH,
    compiler_params=pltpu.CompilerParams(needs_layout_passes=False, use_tc_tiling_on_sc=False),
    scratch_types=[
        pltpu.VMEM_SHARED((NP, EDGE_DIM), _f32),
        pltpu.VMEM_SHARED((NP, EDGE_DIM), _f32),
        pltpu.VMEM((K0, EDGE_DIM), _f32),
        pltpu.VMEM((K0, EDGE_DIM), _f32),
        pltpu.VMEM((K0,), _i32),
    ],
)
def _pass0(*args):
    _pass0_body(*args)


def _passA_body(H, HC, xl_hbm, xr_hbm, ee_hbm, idx3_hbm, att_hbm,
                zrows_hbm, expl, den0, den1,
                dentab, idxb0, idxb1, xlb0, xrb0, eeb0, xlb1, xrb1, eeb1,
                lg0, lg1, attv_ref,
                gx0, gr0, ge0, gx1, gr1, ge1, se0, se1):
    CH16 = HC // LANES
    c = lax.axis_index("c")
    s = lax.axis_index("s")
    w = _worker_id()
    _zero_table(dentab, zrows_hbm)
    pltpu.sync_copy(att_hbm, attv_ref)
    attv = [attv_ref[pl.ds(LANES * j, LANES)] for j in range(CH16)]
    lane = lax.iota(_i32, LANES)
    onehot = [jnp.where(lane == h, 1.0, 0.0).astype(_f32) for h in range(H)]
    base0 = w * T

    def issue(i, idxb, xlb, xrb, eeb, s1, s2, s3):
        pltpu.async_copy(xl_hbm.at[idxb.at[0]], xlb, s1)
        pltpu.async_copy(xr_hbm.at[idxb.at[1]], xrb, s2)
        pltpu.async_copy(ee_hbm.at[pl.ds(base0 + i * KA, KA)], eeb, s3)

    def wait(i, idxb, xlb, xrb, eeb, s1, s2, s3):
        pltpu.make_async_copy(xl_hbm.at[idxb.at[0]], xlb, s1).wait()
        pltpu.make_async_copy(xr_hbm.at[idxb.at[1]], xrb, s2).wait()
        pltpu.make_async_copy(ee_hbm.at[pl.ds(base0 + i * KA, KA)], eeb, s3).wait()

    def compute(i, g, idxb, xlb, xrb, eeb, lg, sste):
        base = base0 + i * KA

        @pl.when(g > 0)
        def _():
            pltpu.make_async_copy(lg, expl.at[pl.ds(base, KA)], sste).wait()

        @plsc.parallel_loop(0, KA, unroll=4)
        def edge(k):
            acc = []
            for j in range(CH16):
                sl = pl.ds(LANES * j, LANES)
                v = xlb[k, sl] + xrb[k, sl] + eeb[k, sl]
                v = jnp.maximum(v, 0.2 * v)
                acc.append(v * attv[j])
            row = jnp.zeros((LANES,), _f32)
            for h in range(H):
                sh = jnp.sum(acc[2 * h] + acc[2 * h + 1])
                row = row + onehot[h] * sh
            ex = jnp.exp(row)
            ex = jnp.where(base + k < E2, ex, jnp.zeros((LANES,), _f32))
            lg[k, :] = ex

        pltpu.async_copy(lg, expl.at[pl.ds(base, KA)], sste)
        pltpu.sync_copy(lg, dentab.at[idxb.at[1]], add=True)

    pltpu.sync_copy(idx3_hbm.at[w, 0], idxb0)
    pltpu.sync_copy(idx3_hbm.at[w, 1], idxb1)
    issue(0, idxb0, xlb0, xrb0, eeb0, gx0, gr0, ge0)

    def pair(g, _):
        i0 = 2 * g
        i1 = i0 + 1
        issue(i1, idxb1, xlb1, xrb1, eeb1, gx1, gr1, ge1)
        wait(i0, idxb0, xlb0, xrb0, eeb0, gx0, gr0, ge0)
        compute(i0, g, idxb0, xlb0, xrb0, eeb0, lg0, se0)

        @pl.when(i0 + 2 < NCHA)
        def _():
            pltpu.sync_copy(idx3_hbm.at[w, i0 + 2], idxb0)
            issue(i0 + 2, idxb0, xlb0, xrb0, eeb0, gx0, gr0, ge0)

        wait(i1, idxb1, xlb1, xrb1, eeb1, gx1, gr1, ge1)
        compute(i1, g, idxb1, xlb1, xrb1, eeb1, lg1, se1)

        @pl.when(i1 + 2 < NCHA)
        def _():
            pltpu.sync_copy(idx3_hbm.at[w, i1 + 2], idxb1)

        return 0

    lax.fori_loop(0, NCHA // 2, pair, 0)
    pltpu.make_async_copy(lg0, expl.at[pl.ds(base0, KA)], se0).wait()
    pltpu.make_async_copy(lg1, expl.at[pl.ds(base0, KA)], se1).wait()
    plsc.subcore_barrier()
    rb = s * RPT

    @pl.when(c == 0)
    def _():
        pltpu.sync_copy(dentab.at[pl.ds(rb, RPT)], den0.at[pl.ds(rb, RPT)])

    @pl.when(c == 1)
    def _():
        pltpu.sync_copy(dentab.at[pl.ds(rb, RPT)], den1.at[pl.ds(rb, RPT)])


def _make_passA(H, HC):
    return pl.kernel(
        functools.partial(_passA_body, H, HC),
        out_type=(jax.ShapeDtypeStruct((E2P, P), _f32),
                  jax.ShapeDtypeStruct((NP, P), _f32),
                  jax.ShapeDtypeStruct((NP, P), _f32)),
        mesh=_MESH,
        compiler_params=pltpu.CompilerParams(needs_layout_passes=False, use_tc_tiling_on_sc=False),
        scratch_types=(
            [pltpu.VMEM_SHARED((NP, P), _f32),
             pltpu.VMEM((2, KA), _i32),
             pltpu.VMEM((2, KA), _i32)]
            + [pltpu.VMEM((KA, HC), _f32)] * 6
            + [pltpu.VMEM((KA, P), _f32)] * 2
            + [pltpu.VMEM((HC,), _f32)]
            + [pltpu.SemaphoreType.DMA] * 8
        ),
    )


def _passB_body(H, HC, xl_hbm, idx3_hbm, expl_hbm, den_hbm,
                zrows_hbm, out0, out1,
                outtab, idxb0, idxb1, xlb0, elg0, db0,
                xlb1, elg1, db1, msg,
                ga0, gb0, gc0, ga1, gb1, gc1):
    CH16 = HC // LANES
    c = lax.axis_index("c")
    s = lax.axis_index("s")
    w = _worker_id()
    _zero_table(outtab, zrows_hbm)
    base0 = w * T

    def issue(i, idxb, xlb, elg, db, s1, s2, s3):
        pltpu.async_copy(xl_hbm.at[idxb.at[0]], xlb, s1)
        pltpu.async_copy(expl_hbm.at[pl.ds(base0 + i * KB, KB)], elg, s2)
        pltpu.async_copy(den_hbm.at[idxb.at[1]], db, s3)

    def wait(i, idxb, xlb, elg, db, s1, s2, s3):
        pltpu.make_async_copy(xl_hbm.at[idxb.at[0]], xlb, s1).wait()
        pltpu.make_async_copy(expl_hbm.at[pl.ds(base0 + i * KB, KB)], elg, s2).wait()
        pltpu.make_async_copy(den_hbm.at[idxb.at[1]], db, s3).wait()

    def compute(i, idxb, xlb, elg, db):
        @plsc.parallel_loop(0, KB, unroll=4)
        def edge(k):
            wv = elg[k, :] / db[k, :]
            for j in range(CH16):
                sl = pl.ds(LANES * j, LANES)
                sp = jnp.full((LANES,), wv[j // 2 if H > 1 else 0], _f32)
                msg[k, sl] = xlb[k, sl] * sp

        pltpu.sync_copy(msg, outtab.at[idxb.at[1]], add=True)

    pltpu.sync_copy(idx3_hbm.at[w, 0], idxb0)
    pltpu.sync_copy(idx3_hbm.at[w, 1], idxb1)
    issue(0, idxb0, xlb0, elg0, db0, ga0, gb0, gc0)

    def pair(g, _):
        i0 = 2 * g
        i1 = i0 + 1
        issue(i1, idxb1, xlb1, elg1, db1, ga1, gb1, gc1)
        wait(i0, idxb0, xlb0, elg0, db0, ga0, gb0, gc0)
        compute(i0, idxb0, xlb0, elg0, db0)

        @pl.when(i0 + 2 < NCHB)
        def _():
            pltpu.sync_copy(idx3_hbm.at[w, i0 + 2], idxb0)
            issue(i0 + 2, idxb0, xlb0, elg0, db0, ga0, gb0, gc0)

        wait(i1, idxb1, xlb1, elg1, db1, ga1, gb1, gc1)
        compute(i1, idxb1, xlb1, elg1, db1)

        @pl.when(i1 + 2 < NCHB)
        def _():
            pltpu.sync_copy(idx3_hbm.at[w, i1 + 2], idxb1)

        return 0

    lax.fori_loop(0, NCHB // 2, pair, 0)
    plsc.subcore_barrier()
    rb = s * RPT

    @pl.when(c == 0)
    def _():
        pltpu.sync_copy(outtab.at[pl.ds(rb, RPT)], out0.at[pl.ds(rb, RPT)])

    @pl.when(c == 1)
    def _():
        pltpu.sync_copy(outtab.at[pl.ds(rb, RPT)], out1.at[pl.ds(rb, RPT)])


def _make_passB(H, HC):
    return pl.kernel(
        functools.partial(_passB_body, H, HC),
        out_type=(jax.ShapeDtypeStruct((NP, HC), _f32),
                  jax.ShapeDtypeStruct((NP, HC), _f32)),
        mesh=_MESH,
        compiler_params=pltpu.CompilerParams(needs_layout_passes=False, use_tc_tiling_on_sc=False),
        scratch_types=(
            [pltpu.VMEM_SHARED((NP, HC), _f32),
             pltpu.VMEM((2, KB), _i32),
             pltpu.VMEM((2, KB), _i32)]
            + [pltpu.VMEM((KB, HC), _f32), pltpu.VMEM((KB, P), _f32),
               pltpu.VMEM((KB, P), _f32)] * 2
            + [pltpu.VMEM((KB, HC), _f32)]
            + [pltpu.SemaphoreType.DMA] * 6
        ),
    )


_PASSA = {(4, 128): _make_passA(4, 128), (1, 32): _make_passA(1, 32)}
_PASSB = {(4, 128): _make_passB(4, 128), (1, 32): _make_passB(1, 32)}


def kernel(x, edge_index, edge_attr, batch, params):
    src = edge_index[0].astype(_i32)
    dst = edge_index[1].astype(_i32)
    batch_i32 = batch.astype(_i32)

    zrows16 = jnp.zeros((RPT, P), _f32)
    ones0 = jnp.zeros((K0, EDGE_DIM), _f32).at[:, 0].set(1.0)

    ea0, ea1, dg0, dg1 = _pass0(edge_attr, dst, ones0, jnp.zeros((RPT, EDGE_DIM), _f32))
    loop_attr = _loopattr(ea0, ea1, dg0, dg1)

    loop = jnp.arange(N, dtype=_i32)
    padi = jnp.zeros((E2P - E2,), _i32)
    src2 = jnp.concatenate([src, loop, padi])
    dst2 = jnp.concatenate([dst, loop, padi])
    idx3a = jnp.concatenate([src2.reshape(NW, NCHA, 1, KA),
                             dst2.reshape(NW, NCHA, 1, KA)], axis=2)
    idx3b = jnp.concatenate([src2.reshape(NW, NCHB, 1, KB),
                             dst2.reshape(NW, NCHB, 1, KB)], axis=2)
    ea2 = jnp.concatenate([edge_attr, loop_attr,
                           jnp.zeros((E2P - E2, EDGE_DIM), _f32)], axis=0)

    h = _dense(x, params["in_W"], params["in_b"])
    for i, (cin, H, C, concat) in enumerate(CFGS):
        HC = H * C
        p = params["gat"][i]
        h_res = h
        xl = _dense(h, p["Wl"], p["bl"])
        xr = _dense(h, p["Wr"], p["br"])
        eemb = _dense(ea2, p["We"], jnp.zeros((HC,), _f32), row_block=4096)
        attf = p["att"].reshape(-1)
        expl, den0, den1 = _PASSA[(H, HC)](xl, xr, eemb, idx3a, attf, zrows16)
        den = _denmerge(den0, den1)
        out0, out1 = _PASSB[(H, HC)](xl, idx3b, expl, den,
                                     jnp.zeros((RPT, HC), _f32))
        h = _post(out0, out1, p["bias"], h_res, params["res"][i],
                  params["ln"][i]["g"], params["ln"][i]["b"])

    return _pool_head(h, batch_i32, params)


# speedup vs baseline: 1.0621x; 1.0459x over previous
import functools

import jax
import jax.numpy as jnp
from jax import lax
from jax.experimental import pallas as pl
from jax.experimental.pallas import tpu as pltpu
from jax.experimental.pallas import tpu_sc as plsc

N = 10000
E = 320000
IN_DIM = 128
EDGE_DIM = 16
HID = 32
EMB = 128
NG = 16
CFGS = [(HID, 4, HID, True), (HID * 4, 4, HID, True), (HID * 4, 4, HID, True), (HID * 4, 1, HID, False)]

NC = 2
NS = 16
NW = NC * NS
LANES = 16
P = 16
K = 128
E2 = E + N
T = ((E2 + NW * K - 1) // (NW * K)) * K
E2P = T * NW
NP = 10240
RPT = NP // NS
K0 = 80
T0 = E // NW

_MESH = plsc.VectorSubcoreMesh(core_axis_name="c", subcore_axis_name="s",
                               num_cores=NC, num_subcores=NS)

_f32 = jnp.float32
_i32 = jnp.int32


def _matmul_bias_kernel(x_ref, w_ref, b_ref, o_ref):
    o_ref[...] = jnp.dot(x_ref[...], w_ref[...],
                         preferred_element_type=_f32) + b_ref[...]


def _dense(x, W, b, row_block=None):
    m, k = x.shape
    n = W.shape[1]
    if row_block is None:
        return pl.pallas_call(
            _matmul_bias_kernel,
            out_shape=jax.ShapeDtypeStruct((m, n), _f32),
        )(x, W, b[None, :])
    assert m % row_block == 0
    return pl.pallas_call(
        _matmul_bias_kernel,
        grid=(m // row_block,),
        in_specs=[
            pl.BlockSpec((row_block, k), lambda i: (i, 0)),
            pl.BlockSpec((k, n), lambda i: (0, 0)),
            pl.BlockSpec((1, n), lambda i: (0, 0)),
        ],
        out_specs=pl.BlockSpec((row_block, n), lambda i: (i, 0)),
        out_shape=jax.ShapeDtypeStruct((m, n), _f32),
    )(x, W, b[None, :])


def _loopattr_kernel(ea0_ref, ea1_ref, dg0_ref, dg1_ref, o_ref):
    deg = dg0_ref[0:N, 0:1] + dg1_ref[0:N, 0:1]
    o_ref[...] = (ea0_ref[0:N] + ea1_ref[0:N]) / jnp.maximum(deg, 1.0)


def _loopattr(ea0, ea1, dg0, dg1):
    return pl.pallas_call(
        _loopattr_kernel,
        out_shape=jax.ShapeDtypeStruct((N, EDGE_DIM), _f32),
    )(ea0, ea1, dg0, dg1)


def _gelu(v):
    return v * 0.5 * (1.0 + lax.erf(v * 0.7071067811865476))


def _ln_gelu(v, g, b):
    mu = jnp.mean(v, axis=-1, keepdims=True)
    var = jnp.mean((v - mu) ** 2, axis=-1, keepdims=True)
    return _gelu((v - mu) / jnp.sqrt(var + 1e-5) * g + b)


def _post_id_kernel(o0_ref, o1_ref, bias_ref, res_ref, g_ref, b_ref, o_ref):
    v = o0_ref[0:N] + o1_ref[0:N] + bias_ref[...] + res_ref[...]
    o_ref[...] = _ln_gelu(v, g_ref[...], b_ref[...])


def _post_proj_kernel(o0_ref, o1_ref, bias_ref, res_ref, w_ref, rb_ref,
                      g_ref, b_ref, o_ref):
    r = jnp.dot(res_ref[...], w_ref[...], preferred_element_type=_f32) + rb_ref[...]
    v = o0_ref[0:N] + o1_ref[0:N] + bias_ref[...] + r
    o_ref[...] = _ln_gelu(v, g_ref[...], b_ref[...])


def _post(out0, out1, bias, h_res, rp, g, b):
    n = out0.shape[1]
    if rp is None:
        return pl.pallas_call(
            _post_id_kernel,
            out_shape=jax.ShapeDtypeStruct((N, n), _f32),
        )(out0, out1, bias[None, :], h_res, g[None, :], b[None, :])
    return pl.pallas_call(
        _post_proj_kernel,
        out_shape=jax.ShapeDtypeStruct((N, n), _f32),
    )(out0, out1, bias[None, :], h_res, rp["W"], rp["b"][None, :],
      g[None, :], b[None, :])


def _denmerge_kernel(d0_ref, d1_ref, o_ref):
    o_ref[...] = d0_ref[...] + d1_ref[...]


def _denmerge(d0, d1):
    return pl.pallas_call(
        _denmerge_kernel,
        out_shape=jax.ShapeDtypeStruct((NP, P), _f32),
    )(d0, d1)


def _pool_head_kernel(h_ref, batch_ref, apw_ref, apb_ref,
                      o1w_ref, o1b_ref, o2w_ref, o2b_ref, o_ref):
    h = h_ref[...]
    att = jax.nn.sigmoid(jnp.dot(h, apw_ref[...],
                                 preferred_element_type=_f32) + apb_ref[...])
    hw = h * att
    onehot = (batch_ref[...] == lax.broadcasted_iota(_i32, (N, NG), 1)
              ).astype(_f32)
    pooled = jnp.dot(onehot.T, hw, preferred_element_type=_f32)
    e = jnp.maximum(jnp.dot(pooled, o1w_ref[...],
                            preferred_element_type=_f32) + o1b_ref[...], 0.0)
    e = jnp.dot(e, o2w_ref[...], preferred_element_type=_f32) + o2b_ref[...]
    nrm = jnp.sqrt(jnp.sum(e * e, axis=-1, keepdims=True))
    o_ref[...] = e / jnp.maximum(nrm, 1e-12)


def _pool_head(h, batch_i32, p):
    return pl.pallas_call(
        _pool_head_kernel,
        out_shape=jax.ShapeDtypeStruct((NG, EMB), _f32),
    )(h, batch_i32[:, None], p["ap_W"], p["ap_b"][None, :],
      p["o1_W"], p["o1_b"][None, :], p["o2_W"], p["o2_b"][None, :])


def _worker_id():
    return lax.axis_index("c") * NS + lax.axis_index("s")


def _zero_table(tab, zrows_hbm):
    s = lax.axis_index("s")
    pltpu.sync_copy(zrows_hbm, tab.at[pl.ds(s * RPT, RPT)])
    plsc.subcore_barrier()


def _pass0_body(ea_hbm, dst_hbm, ones_hbm, zrows_hbm,
                ea0, ea1, dg0, dg1,
                eatab, degtab, eab, onesb, didx):
    c = lax.axis_index("c")
    s = lax.axis_index("s")
    w = _worker_id()
    _zero_table(eatab, zrows_hbm)
    _zero_table(degtab, zrows_hbm)
    pltpu.sync_copy(ones_hbm, onesb)
    base0 = w * T0

    def chunk(i, _):
        base = base0 + i * K0
        pltpu.sync_copy(dst_hbm.at[pl.ds(base, K0)], didx)
        pltpu.sync_copy(ea_hbm.at[pl.ds(base, K0)], eab)
        pltpu.sync_copy(eab, eatab.at[didx], add=True)
        pltpu.sync_copy(onesb, degtab.at[didx], add=True)
        return 0

    lax.fori_loop(0, T0 // K0, chunk, 0)
    plsc.subcore_barrier()
    rb = s * RPT

    @pl.when(c == 0)
    def _():
        pltpu.sync_copy(eatab.at[pl.ds(rb, RPT)], ea0.at[pl.ds(rb, RPT)])
        pltpu.sync_copy(degtab.at[pl.ds(rb, RPT)], dg0.at[pl.ds(rb, RPT)])

    @pl.when(c == 1)
    def _():
        pltpu.sync_copy(eatab.at[pl.ds(rb, RPT)], ea1.at[pl.ds(rb, RPT)])
        pltpu.sync_copy(degtab.at[pl.ds(rb, RPT)], dg1.at[pl.ds(rb, RPT)])


@functools.partial(
    pl.kernel,
    out_type=(jax.ShapeDtypeStruct((NP, EDGE_DIM), _f32),
              jax.ShapeDtypeStruct((NP, EDGE_DIM), _f32),
              jax.ShapeDtypeStruct((NP, EDGE_DIM), _f32),
              jax.ShapeDtypeStruct((NP, EDGE_DIM), _f32)),
    mesh=_MESH,
    compiler_params=pltpu.CompilerParams(needs_layout_passes=False, use_tc_tiling_on_sc=False),
    scratch_types=[
        pltpu.VMEM_SHARED((NP, EDGE_DIM), _f32),
        pltpu.VMEM_SHARED((NP, EDGE_DIM), _f32),
        pltpu.VMEM((K0, EDGE_DIM), _f32),
        pltpu.VMEM((K0, EDGE_DIM), _f32),
        pltpu.VMEM((K0,), _i32),
    ],
)
def _pass0(*args):
    _pass0_body(*args)


def _passA_body(H, HC, xl_hbm, xr_hbm, ee_hbm, src_hbm, dst_hbm, att_hbm,
                zrows_hbm, expl, den0, den1,
                dentab, sidx, didx, xlb, xrb, eeb, lg, attv_ref,
                sem1, sem2, sem3):
    CH16 = HC // LANES
    c = lax.axis_index("c")
    s = lax.axis_index("s")
    w = _worker_id()
    _zero_table(dentab, zrows_hbm)
    pltpu.sync_copy(att_hbm, attv_ref)
    attv = [attv_ref[pl.ds(LANES * j, LANES)] for j in range(CH16)]
    lane = lax.iota(_i32, LANES)
    onehot = [jnp.where(lane == h, 1.0, 0.0).astype(_f32) for h in range(H)]
    base0 = w * T

    def chunk(i, _):
        base = base0 + i * K
        pltpu.sync_copy(src_hbm.at[pl.ds(base, K)], sidx)
        pltpu.sync_copy(dst_hbm.at[pl.ds(base, K)], didx)
        cp1 = pltpu.async_copy(xl_hbm.at[sidx], xlb, sem1)
        cp2 = pltpu.async_copy(xr_hbm.at[didx], xrb, sem2)
        cp3 = pltpu.async_copy(ee_hbm.at[pl.ds(base, K)], eeb, sem3)
        cp1.wait()
        cp2.wait()
        cp3.wait()

        def edge(k, _):
            acc = []
            for j in range(CH16):
                sl = pl.ds(LANES * j, LANES)
                v = xlb[k, sl] + xrb[k, sl] + eeb[k, sl]
                v = jnp.maximum(v, 0.2 * v)
                acc.append(v * attv[j])
            row = jnp.zeros((LANES,), _f32)
            for h in range(H):
                sh = jnp.sum(acc[2 * h] + acc[2 * h + 1])
                row = row + onehot[h] * sh
            ex = jnp.exp(row)
            ex = jnp.where(base + k < E2, ex, jnp.zeros((LANES,), _f32))
            lg[k, :] = ex
            return 0

        lax.fori_loop(0, K, edge, 0, unroll=2)
        pltpu.sync_copy(lg, expl.at[pl.ds(base, K)])
        pltpu.sync_copy(lg, dentab.at[didx], add=True)
        return 0

    lax.fori_loop(0, T // K, chunk, 0)
    plsc.subcore_barrier()
    rb = s * RPT

    @pl.when(c == 0)
    def _():
        pltpu.sync_copy(dentab.at[pl.ds(rb, RPT)], den0.at[pl.ds(rb, RPT)])

    @pl.when(c == 1)
    def _():
        pltpu.sync_copy(dentab.at[pl.ds(rb, RPT)], den1.at[pl.ds(rb, RPT)])


def _make_passA(H, HC):
    return pl.kernel(
        functools.partial(_passA_body, H, HC),
        out_type=(jax.ShapeDtypeStruct((E2P, P), _f32),
                  jax.ShapeDtypeStruct((NP, P), _f32),
                  jax.ShapeDtypeStruct((NP, P), _f32)),
        mesh=_MESH,
        compiler_params=pltpu.CompilerParams(needs_layout_passes=False, use_tc_tiling_on_sc=False),
        scratch_types=[
            pltpu.VMEM_SHARED((NP, P), _f32),
            pltpu.VMEM((K,), _i32),
            pltpu.VMEM((K,), _i32),
            pltpu.VMEM((K, HC), _f32),
            pltpu.VMEM((K, HC), _f32),
            pltpu.VMEM((K, HC), _f32),
            pltpu.VMEM((K, P), _f32),
            pltpu.VMEM((HC,), _f32),
            pltpu.SemaphoreType.DMA,
            pltpu.SemaphoreType.DMA,
            pltpu.SemaphoreType.DMA,
        ],
    )


def _passB_body(H, HC, xl_hbm, src_hbm, dst_hbm, expl_hbm, d0_hbm, d1_hbm,
                zrows_hbm, out0, out1,
                outtab, sidx, didx, xlb, msg, elg, d0b, d1b,
                sem1, sem2, sem3, sem4):
    CH16 = HC // LANES
    c = lax.axis_index("c")
    s = lax.axis_index("s")
    w = _worker_id()
    _zero_table(outtab, zrows_hbm)
    base0 = w * T

    def chunk(i, _):
        base = base0 + i * K
        pltpu.sync_copy(src_hbm.at[pl.ds(base, K)], sidx)
        pltpu.sync_copy(dst_hbm.at[pl.ds(base, K)], didx)
        cp1 = pltpu.async_copy(xl_hbm.at[sidx], xlb, sem1)
        cp2 = pltpu.async_copy(expl_hbm.at[pl.ds(base, K)], elg, sem2)
        cp3 = pltpu.async_copy(d0_hbm.at[didx], d0b, sem3)
        cp4 = pltpu.async_copy(d1_hbm.at[didx], d1b, sem4)
        cp1.wait()
        cp2.wait()
        cp3.wait()
        cp4.wait()

        def edge(k, _):
            dd = d0b[k, :] + d1b[k, :]
            wv = elg[k, :] / dd
            for j in range(CH16):
                sl = pl.ds(LANES * j, LANES)
                sp = jnp.full((LANES,), wv[j // 2 if H > 1 else 0], _f32)
                msg[k, sl] = xlb[k, sl] * sp
            return 0

        lax.fori_loop(0, K, edge, 0, unroll=2)
        pltpu.sync_copy(msg, outtab.at[didx], add=True)
        return 0

    lax.fori_loop(0, T // K, chunk, 0)
    plsc.subcore_barrier()
    rb = s * RPT

    @pl.when(c == 0)
    def _():
        pltpu.sync_copy(outtab.at[pl.ds(rb, RPT)], out0.at[pl.ds(rb, RPT)])

    @pl.when(c == 1)
    def _():
        pltpu.sync_copy(outtab.at[pl.ds(rb, RPT)], out1.at[pl.ds(rb, RPT)])


def _make_passB(H, HC):
    return pl.kernel(
        functools.partial(_passB_body, H, HC),
        out_type=(jax.ShapeDtypeStruct((NP, HC), _f32),
                  jax.ShapeDtypeStruct((NP, HC), _f32)),
        mesh=_MESH,
        compiler_params=pltpu.CompilerParams(needs_layout_passes=False, use_tc_tiling_on_sc=False),
        scratch_types=[
            pltpu.VMEM_SHARED((NP, HC), _f32),
            pltpu.VMEM((K,), _i32),
            pltpu.VMEM((K,), _i32),
            pltpu.VMEM((K, HC), _f32),
            pltpu.VMEM((K, HC), _f32),
            pltpu.VMEM((K, P), _f32),
            pltpu.VMEM((K, P), _f32),
            pltpu.VMEM((K, P), _f32),
            pltpu.SemaphoreType.DMA,
            pltpu.SemaphoreType.DMA,
            pltpu.SemaphoreType.DMA,
            pltpu.SemaphoreType.DMA,
        ],
    )


_PASSA = {(4, 128): _make_passA(4, 128), (1, 32): _make_passA(1, 32)}
_PASSB = {(4, 128): _make_passB(4, 128), (1, 32): _make_passB(1, 32)}


def kernel(x, edge_index, edge_attr, batch, params):
    src = edge_index[0].astype(_i32)
    dst = edge_index[1].astype(_i32)
    batch_i32 = batch.astype(_i32)

    zrows16 = jnp.zeros((RPT, P), _f32)
    ones0 = jnp.zeros((K0, EDGE_DIM), _f32).at[:, 0].set(1.0)

    ea0, ea1, dg0, dg1 = _pass0(edge_attr, dst, ones0, jnp.zeros((RPT, EDGE_DIM), _f32))
    loop_attr = _loopattr(ea0, ea1, dg0, dg1)

    loop = jnp.arange(N, dtype=_i32)
    padi = jnp.zeros((E2P - E2,), _i32)
    src2 = jnp.concatenate([src, loop, padi])
    dst2 = jnp.concatenate([dst, loop, padi])

    ea2 = jnp.concatenate([edge_attr, loop_attr,
                           jnp.zeros((E2P - E2, EDGE_DIM), _f32)], axis=0)

    h = _dense(x, params["in_W"], params["in_b"])
    for i, (cin, H, C, concat) in enumerate(CFGS):
        HC = H * C
        p = params["gat"][i]
        h_res = h
        xl = _dense(h, p["Wl"], p["bl"])
        xr = _dense(h, p["Wr"], p["br"])
        eemb = _dense(ea2, p["We"], jnp.zeros((HC,), _f32), row_block=4096)
        attf = p["att"].reshape(-1)
        expl, den0, den1 = _PASSA[(H, HC)](xl, xr, eemb, src2, dst2, attf, zrows16)
        out0, out1 = _PASSB[(H, HC)](xl, src2, dst2, expl, den0, den1,
                                     jnp.zeros((RPT, HC), _f32))
        h = _post(out0, out1, p["bias"], h_res, params["res"][i],
                  params["ln"][i]["g"], params["ln"][i]["b"])

    return _pool_head(h, batch_i32, params)
